# baseline (speedup 1.0000x reference)
import functools

import jax
import jax.numpy as jnp
import numpy as np
from jax import lax
from jax.experimental import pallas as pl
from jax.experimental.pallas import tpu as pltpu
from jax.experimental.pallas import tpu_sc as plsc

MAX_SEQ_LEN = 512

GATHER_W = 128
CHUNK = 1024


def _pe_table(seq_len: int, dim: int) -> np.ndarray:
    position = np.arange(MAX_SEQ_LEN, dtype=np.float32)[:, None]
    div_term = np.exp(
        np.arange(0, dim, 2, dtype=np.float32) * -(np.log(10000.0) / dim)
    )
    pe = np.zeros((MAX_SEQ_LEN, dim), dtype=np.float32)
    pe[:, 0::2] = np.sin(position * div_term)
    pe[:, 1::2] = np.cos(position * div_term)
    return pe[:seq_len]


@functools.partial(jax.jit, static_argnames=("seq_len",))
def _sc_lookup(idx2, table, pe, seq_len):
    n_rows = idx2.shape[0] * idx2.shape[1]
    dim = table.shape[1]

    info = plsc.get_sparse_core_info()
    nc, ns = info.num_cores, info.num_subcores
    nw = nc * ns
    rows_per_w = n_rows // nw
    assert rows_per_w * nw == n_rows
    assert rows_per_w % CHUNK == 0
    assert rows_per_w % seq_len == 0
    n_chunks = rows_per_w // CHUNK
    g_per_chunk = CHUNK // GATHER_W
    q_regs = dim // 16

    mesh = plsc.VectorSubcoreMesh(core_axis_name="c", subcore_axis_name="s")

    @functools.partial(
        pl.kernel,
        mesh=mesh,
        compiler_params=pltpu.CompilerParams(use_tc_tiling_on_sc=False),
        out_type=jax.ShapeDtypeStruct((n_rows, dim), jnp.float32),
        scratch_types=[
            pltpu.VMEM((g_per_chunk, GATHER_W), jnp.int32),
            pltpu.VMEM((CHUNK, dim), jnp.float32),
            pltpu.VMEM((seq_len, dim), jnp.float32),
            pltpu.SemaphoreType.DMA,
        ],
    )
    def k(idx_hbm, table_hbm, pe_hbm, out_hbm, idx_v, rows_v, pe_v, sem):
        wid = lax.axis_index("s") * nc + lax.axis_index("c")
        pltpu.sync_copy(pe_hbm, pe_v)
        idx_row0 = wid * (rows_per_w // GATHER_W)
        row0 = wid * rows_per_w

        def chunk_body(g, carry):
            pltpu.sync_copy(
                idx_hbm.at[pl.ds(idx_row0 + g * g_per_chunk, g_per_chunk)],
                idx_v,
            )
            copies = []
            for j in range(g_per_chunk):
                copies.append(
                    pltpu.async_copy(
                        table_hbm.at[idx_v.at[j]],
                        rows_v.at[pl.ds(j * GATHER_W, GATHER_W)],
                        sem,
                    )
                )
            for c in copies:
                c.wait()

            base_pos = lax.rem(g * CHUNK, seq_len)

            def pe_body(s, _):
                p = lax.rem(base_pos + s, seq_len)
                for q in range(q_regs):
                    sl = pl.ds(q * 16, 16)
                    rows_v[s, sl] = rows_v[s, sl] + pe_v[p, sl]
                return 0

            lax.fori_loop(0, CHUNK, pe_body, 0)
            pltpu.sync_copy(rows_v, out_hbm.at[pl.ds(row0 + g * CHUNK, CHUNK)])
            return carry

        lax.fori_loop(0, n_chunks, chunk_body, 0)

    return k(idx2, table, pe)


def kernel(input_ids, table):
    batch, seq_len = input_ids.shape
    dim = table.shape[1]
    idx2 = input_ids.astype(jnp.int32).reshape(-1, GATHER_W)
    pe = jnp.asarray(_pe_table(seq_len, dim))
    flat = _sc_lookup(idx2, table, pe, seq_len)
    return (flat.reshape(batch, seq_len, dim), input_ids)


# speedup vs baseline: 1.1427x; 1.1427x over previous
import functools

import jax
import jax.numpy as jnp
import numpy as np
from jax import lax
from jax.experimental import pallas as pl
from jax.experimental.pallas import tpu as pltpu
from jax.experimental.pallas import tpu_sc as plsc

MAX_SEQ_LEN = 512

GATHER_W = 100
CHUNK = 400
NBUF = 3


def _pe_table(seq_len: int, dim: int) -> np.ndarray:
    position = np.arange(MAX_SEQ_LEN, dtype=np.float32)[:, None]
    div_term = np.exp(
        np.arange(0, dim, 2, dtype=np.float32) * -(np.log(10000.0) / dim)
    )
    pe = np.zeros((MAX_SEQ_LEN, dim), dtype=np.float32)
    pe[:, 0::2] = np.sin(position * div_term)
    pe[:, 1::2] = np.cos(position * div_term)
    return pe[:seq_len]


@functools.partial(jax.jit, static_argnames=("seq_len",))
def _sc_lookup(idx2, table, pe, seq_len):
    n_rows = idx2.shape[0] * idx2.shape[1]
    dim = table.shape[1]

    info = plsc.get_sparse_core_info()
    nc, ns = info.num_cores, info.num_subcores
    nw = nc * ns
    rows_per_w = n_rows // nw
    assert rows_per_w * nw == n_rows
    assert rows_per_w % CHUNK == 0
    assert rows_per_w % seq_len == 0
    assert CHUNK % seq_len == 0
    n_chunks = rows_per_w // CHUNK
    g_per_chunk = CHUNK // GATHER_W
    reps = CHUNK // seq_len
    q_regs = dim // 16
    idx_rows_w = rows_per_w // GATHER_W

    mesh = plsc.VectorSubcoreMesh(core_axis_name="c", subcore_axis_name="s")

    @functools.partial(
        pl.kernel,
        mesh=mesh,
        compiler_params=pltpu.CompilerParams(use_tc_tiling_on_sc=False),
        out_type=jax.ShapeDtypeStruct((n_rows, dim), jnp.float32),
        scratch_types=[
            pltpu.VMEM((NBUF, g_per_chunk, GATHER_W), jnp.int32),
            pltpu.VMEM((NBUF, CHUNK, dim), jnp.float32),
            pltpu.VMEM((seq_len, dim), jnp.float32),
            pltpu.SemaphoreType.DMA((NBUF,)),
            pltpu.SemaphoreType.DMA((NBUF,)),
            pltpu.SemaphoreType.DMA((NBUF,)),
        ],
    )
    def k(idx_hbm, table_hbm, pe_hbm, out_hbm, idx_v, rows_v, pe_v,
          sem_i, sem_g, sem_w):
        wid = lax.axis_index("s") * nc + lax.axis_index("c")
        pltpu.sync_copy(pe_hbm, pe_v)
        idx_row0 = wid * idx_rows_w
        row0 = wid * rows_per_w

        def fire_idx(g, buf, sem):
            pltpu.async_copy(
                idx_hbm.at[pl.ds(idx_row0 + g * g_per_chunk, g_per_chunk)],
                idx_v.at[buf], sem)

        def drain_idx(buf, sem):
            pltpu.make_async_copy(
                idx_hbm.at[pl.ds(idx_row0, g_per_chunk)],
                idx_v.at[buf], sem).wait()

        def fire_gathers(buf, sem):
            for j in range(g_per_chunk):
                pltpu.async_copy(
                    table_hbm.at[idx_v.at[buf].at[j]],
                    rows_v.at[buf].at[pl.ds(j * GATHER_W, GATHER_W)],
                    sem)

        def drain_gathers(buf, sem):
            pltpu.make_async_copy(
                table_hbm.at[pl.ds(0, CHUNK)], rows_v.at[buf], sem).wait()

        def fire_out(g, buf, sem):
            pltpu.async_copy(
                rows_v.at[buf], out_hbm.at[pl.ds(row0 + g * CHUNK, CHUNK)],
                sem)

        def drain_out(buf, sem):
            pltpu.make_async_copy(
                rows_v.at[buf], out_hbm.at[pl.ds(row0, CHUNK)], sem).wait()

        fire_idx(0, 0, sem_i.at[0])
        drain_idx(0, sem_i.at[0])
        fire_gathers(0, sem_g.at[0])
        fire_idx(1, 1, sem_i.at[1])
        drain_idx(1, sem_i.at[1])
        fire_gathers(1, sem_g.at[1])
        if n_chunks > 2:
            fire_idx(2, 2, sem_i.at[2])

        def body(g, carry):
            x = lax.rem(g, NBUF)
            nxt2 = lax.rem(g + 2, NBUF)

            @pl.when(g + 3 < n_chunks)
            def _():
                fire_idx(g + 3, lax.rem(g + 3, NBUF), sem_i.at[lax.rem(g + 3, NBUF)])

            drain_gathers(x, sem_g.at[x])

            def srow(s, c):
                for q in range(q_regs):
                    sl = pl.ds(q * 16, 16)
                    pq = pe_v[s, sl]
                    for rep in range(reps):
                        r = rep * seq_len + s
                        rows_v[x, r, sl] = rows_v[x, r, sl] + pq
                return c

            lax.fori_loop(0, seq_len, srow, 0)
            fire_out(g, x, sem_w.at[x])

            @pl.when(g + 2 < n_chunks)
            def _():
                @pl.when(g >= 1)
                def _():
                    drain_out(nxt2, sem_w.at[nxt2])
                drain_idx(nxt2, sem_i.at[nxt2])
                fire_gathers(nxt2, sem_g.at[nxt2])
            return carry

        lax.fori_loop(0, n_chunks, body, 0)
        drain_out(lax.rem(n_chunks - 2, NBUF), sem_w.at[lax.rem(n_chunks - 2, NBUF)])
        drain_out(lax.rem(n_chunks - 1, NBUF), sem_w.at[lax.rem(n_chunks - 1, NBUF)])

    return k(idx2, table, pe)


def kernel(input_ids, table):
    batch, seq_len = input_ids.shape
    dim = table.shape[1]
    idx2 = input_ids.astype(jnp.int32).reshape(-1, GATHER_W)
    pe = jnp.asarray(_pe_table(seq_len, dim))
    flat = _sc_lookup(idx2, table, pe, seq_len)
    return (flat.reshape(batch, seq_len, dim), input_ids)


# speedup vs baseline: 1.3050x; 1.1420x over previous
import functools

import jax
import jax.numpy as jnp
import numpy as np
from jax import lax
from jax.experimental import pallas as pl
from jax.experimental.pallas import tpu as pltpu
from jax.experimental.pallas import tpu_sc as plsc

MAX_SEQ_LEN = 512
PAD_DIM = 128

GATHER_W = 100
CHUNK = 200
NBUF_G = 5
NBUF_O = 2


def _pe_table(seq_len: int, dim: int) -> np.ndarray:
    position = np.arange(MAX_SEQ_LEN, dtype=np.float32)[:, None]
    div_term = np.exp(
        np.arange(0, dim, 2, dtype=np.float32) * -(np.log(10000.0) / dim)
    )
    pe = np.zeros((MAX_SEQ_LEN, dim), dtype=np.float32)
    pe[:, 0::2] = np.sin(position * div_term)
    pe[:, 1::2] = np.cos(position * div_term)
    return pe[:seq_len]


@functools.partial(jax.jit, static_argnames=("seq_len", "dim"))
def _sc_lookup(idx2, table, pe, seq_len, dim):
    n_rows = idx2.shape[0] * idx2.shape[1]

    info = plsc.get_sparse_core_info()
    nc, ns = info.num_cores, info.num_subcores
    nw = nc * ns
    rows_per_w = n_rows // nw
    assert rows_per_w * nw == n_rows
    assert rows_per_w % CHUNK == 0
    assert rows_per_w % seq_len == 0
    assert seq_len % CHUNK == 0
    n_chunks = rows_per_w // CHUNK
    g_per_chunk = CHUNK // GATHER_W
    q_regs = dim // 16
    idx_rows_w = rows_per_w // GATHER_W

    mesh = plsc.VectorSubcoreMesh(core_axis_name="c", subcore_axis_name="s")

    @functools.partial(
        pl.kernel,
        mesh=mesh,
        compiler_params=pltpu.CompilerParams(use_tc_tiling_on_sc=False),
        out_type=jax.ShapeDtypeStruct((n_rows, PAD_DIM), jnp.float32),
        scratch_types=[
            pltpu.VMEM((NBUF_G, g_per_chunk, GATHER_W), jnp.int32),
            pltpu.VMEM((NBUF_G, CHUNK, dim), jnp.float32),
            pltpu.VMEM((NBUF_O, CHUNK, PAD_DIM), jnp.float32),
            pltpu.VMEM((seq_len, dim), jnp.float32),
            pltpu.SemaphoreType.DMA((NBUF_G,)),
            pltpu.SemaphoreType.DMA((NBUF_G,)),
            pltpu.SemaphoreType.DMA((NBUF_O,)),
        ],
    )
    def k(idx_hbm, table_hbm, pe_hbm, out_hbm, idx_v, g_v, o_v, pe_v,
          sem_i, sem_g, sem_w):
        wid = lax.axis_index("s") * nc + lax.axis_index("c")
        pltpu.sync_copy(pe_hbm, pe_v)
        idx_row0 = wid * idx_rows_w
        row0 = wid * rows_per_w

        def fire_idx(g, buf):
            pltpu.async_copy(
                idx_hbm.at[pl.ds(idx_row0 + g * g_per_chunk, g_per_chunk)],
                idx_v.at[buf], sem_i.at[buf])

        def drain_idx(buf):
            pltpu.make_async_copy(
                idx_hbm.at[pl.ds(idx_row0, g_per_chunk)],
                idx_v.at[buf], sem_i.at[buf]).wait()

        def fire_gathers(buf):
            for j in range(g_per_chunk):
                pltpu.async_copy(
                    table_hbm.at[idx_v.at[buf].at[j]],
                    g_v.at[buf].at[pl.ds(j * GATHER_W, GATHER_W)],
                    sem_g.at[buf])

        def drain_gathers(buf):
            pltpu.make_async_copy(
                table_hbm.at[pl.ds(0, CHUNK)], g_v.at[buf],
                sem_g.at[buf]).wait()

        def fire_out(g, buf):
            pltpu.async_copy(
                o_v.at[buf], out_hbm.at[pl.ds(row0 + g * CHUNK, CHUNK)],
                sem_w.at[buf])

        def drain_out(buf):
            pltpu.make_async_copy(
                o_v.at[buf], out_hbm.at[pl.ds(row0, CHUNK)],
                sem_w.at[buf]).wait()

        for p in range(min(NBUF_G - 1, n_chunks)):
            fire_idx(p, p)
            drain_idx(p)
            fire_gathers(p)
        if n_chunks > NBUF_G - 1:
            fire_idx(NBUF_G - 1, NBUF_G - 1)

        def body(g, carry):
            xg = lax.rem(g, NBUF_G)
            xo = lax.rem(g, NBUF_O)
            drain_gathers(xg)

            @pl.when(g + NBUF_G < n_chunks)
            def _():
                fire_idx(g + NBUF_G, xg)

            @pl.when(g >= NBUF_O)
            def _():
                drain_out(xo)

            def srow(s, c):
                for q in range(q_regs):
                    sl = pl.ds(q * 16, 16)
                    o_v[xo, s, sl] = g_v[xg, s, sl] + pe_v[s, sl]
                return c

            lax.fori_loop(0, CHUNK, srow, 0)
            fire_out(g, xo)

            nxt = lax.rem(g + NBUF_G - 1, NBUF_G)

            @pl.when(g + NBUF_G - 1 < n_chunks)
            def _():
                drain_idx(nxt)
                fire_gathers(nxt)
            return carry

        lax.fori_loop(0, n_chunks, body, 0)
        for p in range(min(NBUF_O, n_chunks)):
            drain_out(lax.rem(n_chunks - 1 - p, NBUF_O))

    return k(idx2, table, pe)


def kernel(input_ids, table):
    batch, seq_len = input_ids.shape
    dim = table.shape[1]
    idx2 = input_ids.astype(jnp.int32).reshape(-1, GATHER_W)
    pe = jnp.asarray(_pe_table(seq_len, dim))
    out128 = _sc_lookup(idx2, table, pe, seq_len, dim)
    emb = out128[:, :dim].reshape(batch, seq_len, dim)
    return (emb, input_ids)
